# baseline (speedup 1.0000x reference)
import functools

import jax
import jax.numpy as jnp
from jax import lax
from jax.experimental import pallas as pl
from jax.experimental.pallas import tpu as pltpu
from jax.experimental.pallas import tpu_sc as plsc

N = 10000
E = 320000
D = 128

NC, NS = 2, 16
NW = NC * NS
EPW = E // NW
CHUNK = 80
NCH = EPW // CHUNK
NPAD = 10240
PAD_PT = NPAD // NS
ROWS_PT = PAD_PT

ECHUNK = 40
ENCH = EPW // ECHUNK
ENBUF = 8
GOFF = 2
SOFF = 5
DNBUF = 5

_MESH = plsc.VectorSubcoreMesh(
    core_axis_name="c", subcore_axis_name="s", num_cores=NC, num_subcores=NS)


def _worker_id():
    return lax.axis_index("s") * NC + lax.axis_index("c")


@functools.partial(
    pl.kernel,
    out_type=[
        jax.ShapeDtypeStruct((NC, NPAD), jnp.float32),
        jax.ShapeDtypeStruct((NC, NPAD), jnp.float32),
    ],
    mesh=_MESH,
    scratch_types=[
        pltpu.VMEM((NCH, CHUNK), jnp.int32),
        pltpu.VMEM((NCH, CHUNK), jnp.int32),
        pltpu.VMEM((CHUNK,), jnp.float32),
        pltpu.VMEM((DNBUF, CHUNK), jnp.float32),
        pltpu.VMEM_SHARED((NPAD,), jnp.float32),
        pltpu.VMEM_SHARED((NPAD,), jnp.float32),
        pltpu.SemaphoreType.DMA((DNBUF,)),
        pltpu.SemaphoreType.DMA((DNBUF,)),
    ],
)
def _deg_kernel(row3, col3, z1, cnt_out, sl_out,
                rowm, colm, onesv, slb, cnt_acc, sl_acc, s1, s2):
    cid = lax.axis_index("c")
    sid = lax.axis_index("s")
    wid = _worker_id()
    pltpu.sync_copy(z1, cnt_acc.at[pl.ds(sid * PAD_PT, PAD_PT)])
    pltpu.sync_copy(z1, sl_acc.at[pl.ds(sid * PAD_PT, PAD_PT)])
    pltpu.sync_copy(row3.at[wid], rowm)
    pltpu.sync_copy(col3.at[wid], colm)
    for i in range(CHUNK // 16):
        onesv[pl.ds(16 * i, 16)] = jnp.full((16,), 1.0, jnp.float32)
    plsc.subcore_barrier()

    @pl.loop(0, NCH // DNBUF)
    def _outer(g):
        for b in range(DNBUF):
            j = g * DNBUF + b

            @pl.when(g > 0)
            def _reuse_wait():
                pltpu.make_async_copy(
                    onesv, cnt_acc.at[rowm.at[j]], s1.at[b]).wait()
                pltpu.make_async_copy(
                    slb.at[b], sl_acc.at[rowm.at[j]], s2.at[b]).wait()

            for i in range(CHUNK // 16):
                r = rowm[j, pl.ds(16 * i, 16)]
                c = colm[j, pl.ds(16 * i, 16)]
                slb[b, pl.ds(16 * i, 16)] = jnp.where(
                    r == c, jnp.full((16,), 1.0, jnp.float32),
                    jnp.zeros((16,), jnp.float32))
            pltpu.async_copy(onesv, cnt_acc.at[rowm.at[j]], s1.at[b], add=True)
            pltpu.async_copy(slb.at[b], sl_acc.at[rowm.at[j]], s2.at[b],
                             add=True)

    for b in range(DNBUF):
        pltpu.make_async_copy(onesv, cnt_acc.at[rowm.at[0]], s1.at[b]).wait()
        pltpu.make_async_copy(slb.at[b], sl_acc.at[rowm.at[0]], s2.at[b]).wait()
    plsc.subcore_barrier()
    pltpu.sync_copy(cnt_acc.at[pl.ds(sid * PAD_PT, PAD_PT)],
                    cnt_out.at[cid, pl.ds(sid * PAD_PT, PAD_PT)])
    pltpu.sync_copy(sl_acc.at[pl.ds(sid * PAD_PT, PAD_PT)],
                    sl_out.at[cid, pl.ds(sid * PAD_PT, PAD_PT)])


@functools.partial(
    pl.kernel,
    out_type=jax.ShapeDtypeStruct((NC, NPAD, D), jnp.float32),
    mesh=_MESH,
    scratch_types=(
        [pltpu.VMEM((ECHUNK,), jnp.int32) for _ in range(ENBUF)]
        + [pltpu.VMEM((ECHUNK,), jnp.int32) for _ in range(ENBUF)]
        + [pltpu.VMEM((ECHUNK, D), jnp.float32) for _ in range(ENBUF)]
        + [pltpu.VMEM_SHARED((NPAD, D), jnp.float32),
           pltpu.SemaphoreType.DMA((ENBUF,)),
           pltpu.SemaphoreType.DMA((ENBUF,)),
           pltpu.SemaphoreType.DMA((ENBUF,))]
    ),
)
def _edge_kernel(xp, row3, col3, z2, out, *rest):
    rowf = rest[:ENBUF]
    colf = rest[ENBUF:2 * ENBUF]
    xbs = rest[2 * ENBUF:3 * ENBUF]
    acc, isem, gsem, ssem = rest[3 * ENBUF:]
    cid = lax.axis_index("c")
    sid = lax.axis_index("s")
    wid = _worker_id()
    pltpu.sync_copy(z2, acc.at[pl.ds(sid * ROWS_PT, ROWS_PT)])
    plsc.subcore_barrier()

    @pl.loop(0, (ENCH + SOFF + ENBUF - 1) // ENBUF)
    def _outer(g):
        for b in range(ENBUF):
            j = g * ENBUF + b

            @pl.when((j >= ENBUF) & (j < ENCH))
            def _reuse_wait():
                pltpu.make_async_copy(
                    xbs[b], acc.at[colf[b]], ssem.at[b]).wait()

            @pl.when(j < ENCH)
            def _idx_load():
                pltpu.async_copy(row3.at[wid, j], rowf[b], isem.at[b])
                pltpu.async_copy(col3.at[wid, j], colf[b], isem.at[b])

            b2 = (b - GOFF) % ENBUF
            j2 = j - GOFF

            @pl.when((j2 >= 0) & (j2 < ENCH))
            def _gather():
                pltpu.make_async_copy(row3.at[wid, j2], rowf[b2],
                                      isem.at[b2]).wait()
                pltpu.make_async_copy(col3.at[wid, j2], colf[b2],
                                      isem.at[b2]).wait()
                pltpu.async_copy(xp.at[rowf[b2]], xbs[b2], gsem.at[b2])

            b3 = (b - SOFF) % ENBUF
            j3 = j - SOFF

            @pl.when((j3 >= 0) & (j3 < ENCH))
            def _scatter():
                pltpu.make_async_copy(xp.at[rowf[b3]], xbs[b3],
                                      gsem.at[b3]).wait()
                pltpu.async_copy(xbs[b3], acc.at[colf[b3]], ssem.at[b3],
                                 add=True)

    for b in range(ENBUF):
        pltpu.make_async_copy(xbs[b], acc.at[colf[b]], ssem.at[b]).wait()
    plsc.subcore_barrier()
    pltpu.sync_copy(acc.at[pl.ds(sid * ROWS_PT, ROWS_PT)],
                    out.at[cid, pl.ds(sid * ROWS_PT, ROWS_PT)])


BLK = 1000


def _prep_body(cnt_ref, sl_ref, x_ref, dis_ref, corr_ref, xp_ref):
    cnt = cnt_ref[0] + cnt_ref[1]
    sl = sl_ref[0] + sl_ref[1]
    deg = cnt - sl
    dis = jnp.where(deg > 0, lax.rsqrt(jnp.maximum(deg, 1e-12)), 0.0)
    dis_ref[...] = dis
    corr_ref[...] = sl * dis * dis
    xp_ref[...] = dis * x_ref[...]


_prep_call = pl.pallas_call(
    _prep_body,
    grid=(N // BLK,),
    in_specs=[
        pl.BlockSpec((NC, BLK, 1), lambda i: (0, i, 0)),
        pl.BlockSpec((NC, BLK, 1), lambda i: (0, i, 0)),
        pl.BlockSpec((BLK, D), lambda i: (i, 0)),
    ],
    out_specs=[
        pl.BlockSpec((BLK, 1), lambda i: (i, 0)),
        pl.BlockSpec((BLK, 1), lambda i: (i, 0)),
        pl.BlockSpec((BLK, D), lambda i: (i, 0)),
    ],
    out_shape=[
        jax.ShapeDtypeStruct((N, 1), jnp.float32),
        jax.ShapeDtypeStruct((N, 1), jnp.float32),
        jax.ShapeDtypeStruct((N, D), jnp.float32),
    ],
)


def _dense_body(x_ref, acc_ref, dis_ref, corr_ref, w0_ref, w1_ref, b_ref,
                *out_refs, relu):
    x = x_ref[...]
    dis = dis_ref[...]
    tx = corr_ref[...] * x - dis * (acc_ref[0] + acc_ref[1])
    y = (jnp.dot(x, w0_ref[...], preferred_element_type=jnp.float32)
         + jnp.dot(tx, w1_ref[...], preferred_element_type=jnp.float32)
         + b_ref[...])
    if relu:
        y = jnp.maximum(y, 0.0)
        out_refs[0][...] = y
        out_refs[1][...] = dis * y
    else:
        out_refs[0][...] = y


def _make_dense(relu):
    n_out = 2 if relu else 1
    return pl.pallas_call(
        functools.partial(_dense_body, relu=relu),
        grid=(N // BLK,),
        in_specs=[
            pl.BlockSpec((BLK, D), lambda i: (i, 0)),
            pl.BlockSpec((NC, BLK, D), lambda i: (0, i, 0)),
            pl.BlockSpec((BLK, 1), lambda i: (i, 0)),
            pl.BlockSpec((BLK, 1), lambda i: (i, 0)),
            pl.BlockSpec((D, D), lambda i: (0, 0)),
            pl.BlockSpec((D, D), lambda i: (0, 0)),
            pl.BlockSpec((1, D), lambda i: (0, 0)),
        ],
        out_specs=[pl.BlockSpec((BLK, D), lambda i: (i, 0))] * n_out,
        out_shape=[jax.ShapeDtypeStruct((N, D), jnp.float32)] * n_out,
    )


_dense_relu = _make_dense(True)
_dense_last = _make_dense(False)


def kernel(embedding, W0a, W1a, b1, W0b, W1b, b2, prop_edge_index):
    row = prop_edge_index[0].astype(jnp.int32)
    col = prop_edge_index[1].astype(jnp.int32)
    row3 = row.reshape(NW, NCH, CHUNK)
    col3 = col.reshape(NW, NCH, CHUNK)
    row3e = row.reshape(NW, ENCH, ECHUNK)
    col3e = col.reshape(NW, ENCH, ECHUNK)
    z1 = jnp.zeros((PAD_PT,), jnp.float32)
    z2 = jnp.zeros((ROWS_PT, D), jnp.float32)

    cnt_p, sl_p = _deg_kernel(row3, col3, z1)
    dis, corr, xp1 = _prep_call(cnt_p[:, :N, None], sl_p[:, :N, None],
                                embedding)
    acc1 = _edge_kernel(xp1, row3e, col3e, z2)
    h, xp2 = _dense_relu(embedding, acc1, dis, corr, W0a, W1a,
                         b1.reshape(1, D))
    acc2 = _edge_kernel(xp2, row3e, col3e, z2)
    out, = _dense_last(h, acc2, dis, corr, W0b, W1b, b2.reshape(1, D))
    return out


# speedup vs baseline: 1.0337x; 1.0337x over previous
import functools

import jax
import jax.numpy as jnp
from jax import lax
from jax.experimental import pallas as pl
from jax.experimental.pallas import tpu as pltpu
from jax.experimental.pallas import tpu_sc as plsc

N = 10000
E = 320000
D = 128

NC, NS = 2, 16
NW = NC * NS
EPW = E // NW
CHUNK = 80
NCH = EPW // CHUNK
NPAD = 10240
PAD_PT = NPAD // NS
ROWS_PT = PAD_PT

ECHUNK = 80
ENCH = EPW // ECHUNK
ENBUF = 4
GOFF = 1
SOFF = 3
DNBUF = 5

_MESH = plsc.VectorSubcoreMesh(
    core_axis_name="c", subcore_axis_name="s", num_cores=NC, num_subcores=NS)


def _worker_id():
    return lax.axis_index("s") * NC + lax.axis_index("c")


@functools.partial(
    pl.kernel,
    out_type=[
        jax.ShapeDtypeStruct((NC, NPAD), jnp.float32),
        jax.ShapeDtypeStruct((NC, NPAD), jnp.float32),
    ],
    mesh=_MESH,
    scratch_types=[
        pltpu.VMEM((NCH, CHUNK), jnp.int32),
        pltpu.VMEM((NCH, CHUNK), jnp.int32),
        pltpu.VMEM((CHUNK,), jnp.float32),
        pltpu.VMEM((DNBUF, CHUNK), jnp.float32),
        pltpu.VMEM_SHARED((NPAD,), jnp.float32),
        pltpu.VMEM_SHARED((NPAD,), jnp.float32),
        pltpu.SemaphoreType.DMA((DNBUF,)),
        pltpu.SemaphoreType.DMA((DNBUF,)),
    ],
)
def _deg_kernel(row3, col3, z1, cnt_out, sl_out,
                rowm, colm, onesv, slb, cnt_acc, sl_acc, s1, s2):
    cid = lax.axis_index("c")
    sid = lax.axis_index("s")
    wid = _worker_id()
    pltpu.sync_copy(z1, cnt_acc.at[pl.ds(sid * PAD_PT, PAD_PT)])
    pltpu.sync_copy(z1, sl_acc.at[pl.ds(sid * PAD_PT, PAD_PT)])
    pltpu.sync_copy(row3.at[wid], rowm)
    pltpu.sync_copy(col3.at[wid], colm)
    for i in range(CHUNK // 16):
        onesv[pl.ds(16 * i, 16)] = jnp.full((16,), 1.0, jnp.float32)
    plsc.subcore_barrier()

    @pl.loop(0, NCH // DNBUF)
    def _outer(g):
        for b in range(DNBUF):
            j = g * DNBUF + b

            @pl.when(g > 0)
            def _reuse_wait():
                pltpu.make_async_copy(
                    onesv, cnt_acc.at[rowm.at[j]], s1.at[b]).wait()
                pltpu.make_async_copy(
                    slb.at[b], sl_acc.at[rowm.at[j]], s2.at[b]).wait()

            for i in range(CHUNK // 16):
                r = rowm[j, pl.ds(16 * i, 16)]
                c = colm[j, pl.ds(16 * i, 16)]
                slb[b, pl.ds(16 * i, 16)] = jnp.where(
                    r == c, jnp.full((16,), 1.0, jnp.float32),
                    jnp.zeros((16,), jnp.float32))
            pltpu.async_copy(onesv, cnt_acc.at[rowm.at[j]], s1.at[b], add=True)
            pltpu.async_copy(slb.at[b], sl_acc.at[rowm.at[j]], s2.at[b],
                             add=True)

    for b in range(DNBUF):
        pltpu.make_async_copy(onesv, cnt_acc.at[rowm.at[0]], s1.at[b]).wait()
        pltpu.make_async_copy(slb.at[b], sl_acc.at[rowm.at[0]], s2.at[b]).wait()
    plsc.subcore_barrier()
    pltpu.sync_copy(cnt_acc.at[pl.ds(sid * PAD_PT, PAD_PT)],
                    cnt_out.at[cid, pl.ds(sid * PAD_PT, PAD_PT)])
    pltpu.sync_copy(sl_acc.at[pl.ds(sid * PAD_PT, PAD_PT)],
                    sl_out.at[cid, pl.ds(sid * PAD_PT, PAD_PT)])


@functools.partial(
    pl.kernel,
    out_type=jax.ShapeDtypeStruct((NC, NPAD, D), jnp.float32),
    mesh=_MESH,
    scratch_types=(
        [pltpu.VMEM((ECHUNK,), jnp.int32) for _ in range(ENBUF)]
        + [pltpu.VMEM((ECHUNK,), jnp.int32) for _ in range(ENBUF)]
        + [pltpu.VMEM((ECHUNK, D), jnp.float32) for _ in range(ENBUF)]
        + [pltpu.VMEM_SHARED((NPAD, D), jnp.float32),
           pltpu.SemaphoreType.DMA((ENBUF,)),
           pltpu.SemaphoreType.DMA((ENBUF,)),
           pltpu.SemaphoreType.DMA((ENBUF,))]
    ),
)
def _edge_kernel(xp, row3, col3, z2, out, *rest):
    rowf = rest[:ENBUF]
    colf = rest[ENBUF:2 * ENBUF]
    xbs = rest[2 * ENBUF:3 * ENBUF]
    acc, isem, gsem, ssem = rest[3 * ENBUF:]
    cid = lax.axis_index("c")
    sid = lax.axis_index("s")
    wid = _worker_id()
    pltpu.sync_copy(z2, acc.at[pl.ds(sid * ROWS_PT, ROWS_PT)])
    plsc.subcore_barrier()

    @pl.loop(0, (ENCH + SOFF + ENBUF - 1) // ENBUF)
    def _outer(g):
        for b in range(ENBUF):
            j = g * ENBUF + b

            @pl.when((j >= ENBUF) & (j < ENCH))
            def _reuse_wait():
                pltpu.make_async_copy(
                    xbs[b], acc.at[colf[b]], ssem.at[b]).wait()

            @pl.when(j < ENCH)
            def _idx_load():
                pltpu.async_copy(row3.at[wid, j], rowf[b], isem.at[b])
                pltpu.async_copy(col3.at[wid, j], colf[b], isem.at[b])

            b2 = (b - GOFF) % ENBUF
            j2 = j - GOFF

            @pl.when((j2 >= 0) & (j2 < ENCH))
            def _gather():
                pltpu.make_async_copy(row3.at[wid, j2], rowf[b2],
                                      isem.at[b2]).wait()
                pltpu.make_async_copy(col3.at[wid, j2], colf[b2],
                                      isem.at[b2]).wait()
                pltpu.async_copy(xp.at[rowf[b2]], xbs[b2], gsem.at[b2])

            b3 = (b - SOFF) % ENBUF
            j3 = j - SOFF

            @pl.when((j3 >= 0) & (j3 < ENCH))
            def _scatter():
                pltpu.make_async_copy(xp.at[rowf[b3]], xbs[b3],
                                      gsem.at[b3]).wait()
                pltpu.async_copy(xbs[b3], acc.at[colf[b3]], ssem.at[b3],
                                 add=True)

    for b in range(ENBUF):
        pltpu.make_async_copy(xbs[b], acc.at[colf[b]], ssem.at[b]).wait()
    plsc.subcore_barrier()
    pltpu.sync_copy(acc.at[pl.ds(sid * ROWS_PT, ROWS_PT)],
                    out.at[cid, pl.ds(sid * ROWS_PT, ROWS_PT)])


BLK = 1000


def _prep_body(cnt_ref, sl_ref, x_ref, dis_ref, corr_ref, xp_ref):
    cnt = cnt_ref[0] + cnt_ref[1]
    sl = sl_ref[0] + sl_ref[1]
    deg = cnt - sl
    dis = jnp.where(deg > 0, lax.rsqrt(jnp.maximum(deg, 1e-12)), 0.0)
    dis_ref[...] = dis
    corr_ref[...] = sl * dis * dis
    xp_ref[...] = dis * x_ref[...]


_prep_call = pl.pallas_call(
    _prep_body,
    grid=(N // BLK,),
    in_specs=[
        pl.BlockSpec((NC, BLK, 1), lambda i: (0, i, 0)),
        pl.BlockSpec((NC, BLK, 1), lambda i: (0, i, 0)),
        pl.BlockSpec((BLK, D), lambda i: (i, 0)),
    ],
    out_specs=[
        pl.BlockSpec((BLK, 1), lambda i: (i, 0)),
        pl.BlockSpec((BLK, 1), lambda i: (i, 0)),
        pl.BlockSpec((BLK, D), lambda i: (i, 0)),
    ],
    out_shape=[
        jax.ShapeDtypeStruct((N, 1), jnp.float32),
        jax.ShapeDtypeStruct((N, 1), jnp.float32),
        jax.ShapeDtypeStruct((N, D), jnp.float32),
    ],
)


def _dense_body(x_ref, acc_ref, dis_ref, corr_ref, w0_ref, w1_ref, b_ref,
                *out_refs, relu):
    x = x_ref[...]
    dis = dis_ref[...]
    tx = corr_ref[...] * x - dis * (acc_ref[0] + acc_ref[1])
    y = (jnp.dot(x, w0_ref[...], preferred_element_type=jnp.float32)
         + jnp.dot(tx, w1_ref[...], preferred_element_type=jnp.float32)
         + b_ref[...])
    if relu:
        y = jnp.maximum(y, 0.0)
        out_refs[0][...] = y
        out_refs[1][...] = dis * y
    else:
        out_refs[0][...] = y


def _make_dense(relu):
    n_out = 2 if relu else 1
    return pl.pallas_call(
        functools.partial(_dense_body, relu=relu),
        grid=(N // BLK,),
        in_specs=[
            pl.BlockSpec((BLK, D), lambda i: (i, 0)),
            pl.BlockSpec((NC, BLK, D), lambda i: (0, i, 0)),
            pl.BlockSpec((BLK, 1), lambda i: (i, 0)),
            pl.BlockSpec((BLK, 1), lambda i: (i, 0)),
            pl.BlockSpec((D, D), lambda i: (0, 0)),
            pl.BlockSpec((D, D), lambda i: (0, 0)),
            pl.BlockSpec((1, D), lambda i: (0, 0)),
        ],
        out_specs=[pl.BlockSpec((BLK, D), lambda i: (i, 0))] * n_out,
        out_shape=[jax.ShapeDtypeStruct((N, D), jnp.float32)] * n_out,
    )


_dense_relu = _make_dense(True)
_dense_last = _make_dense(False)


def kernel(embedding, W0a, W1a, b1, W0b, W1b, b2, prop_edge_index):
    row = prop_edge_index[0].astype(jnp.int32)
    col = prop_edge_index[1].astype(jnp.int32)
    row3 = row.reshape(NW, NCH, CHUNK)
    col3 = col.reshape(NW, NCH, CHUNK)
    row3e = row.reshape(NW, ENCH, ECHUNK)
    col3e = col.reshape(NW, ENCH, ECHUNK)
    z1 = jnp.zeros((PAD_PT,), jnp.float32)
    z2 = jnp.zeros((ROWS_PT, D), jnp.float32)

    cnt_p, sl_p = _deg_kernel(row3, col3, z1)
    dis, corr, xp1 = _prep_call(cnt_p[:, :N, None], sl_p[:, :N, None],
                                embedding)
    acc1 = _edge_kernel(xp1, row3e, col3e, z2)
    h, xp2 = _dense_relu(embedding, acc1, dis, corr, W0a, W1a,
                         b1.reshape(1, D))
    acc2 = _edge_kernel(xp2, row3e, col3e, z2)
    out, = _dense_last(h, acc2, dis, corr, W0b, W1b, b2.reshape(1, D))
    return out
